# baseline (speedup 1.0000x reference)
import functools

import jax
import jax.numpy as jnp
from jax import lax
from jax.experimental import pallas as pl
from jax.experimental.pallas import tpu as pltpu
from jax.experimental.pallas import tpu_sc as plsc

_N_SC = 32
_CH = 32
_LANES = 16


def _tc_body(x_ref, w_ref, b_ref, out_ref):
    s = jnp.sum(x_ref[...], axis=1)
    pooled = s * (1.0 / x_ref.shape[1])
    logits = lax.dot_general(
        pooled, w_ref[...], (((1,), (1,)), ((), ())),
        preferred_element_type=jnp.float32,
    ) + b_ref[...]
    bb, e = logits.shape
    idx = lax.broadcasted_iota(jnp.int32, (bb, e), 1)
    m1 = jnp.max(logits, axis=1, keepdims=True)
    i1 = jnp.min(jnp.where(logits == m1, idx, e), axis=1, keepdims=True)
    neg = jnp.where(idx == i1, -jnp.inf, logits)
    m2 = jnp.max(neg, axis=1, keepdims=True)
    i2 = jnp.min(jnp.where(neg == m2, idx, e), axis=1, keepdims=True)
    e2 = jnp.exp(m2 - m1)
    denom = 1.0 + e2
    w1 = 1.0 / denom
    w2 = e2 / denom
    out_ref[...] = jnp.where(idx == i1, w1, jnp.where(idx == i2, w2, 0.0))


def _make_sc_gate(B, HW, C, E, base_row):
    n_chunks = HW // _CH
    n_cgrp = C // _LANES
    n_egrp = E // _LANES
    mesh = plsc.VectorSubcoreMesh(core_axis_name="c", subcore_axis_name="s")

    @functools.partial(
        pl.kernel,
        mesh=mesh,
        out_type=jax.ShapeDtypeStruct((_N_SC, E), jnp.float32),
        scratch_types=[
            pltpu.VMEM((_CH, C), jnp.float32),
            pltpu.VMEM((_CH, C), jnp.float32),
            pltpu.VMEM((C * E,), jnp.float32),
            pltpu.VMEM((E,), jnp.float32),
            pltpu.VMEM((C,), jnp.float32),
            pltpu.VMEM((E,), jnp.float32),
            pltpu.SemaphoreType.DMA,
            pltpu.SemaphoreType.DMA,
            pltpu.SemaphoreType.DMA,
        ],
    )
    def sc_gate(x_hbm, wt_hbm, b_hbm, out_hbm,
                buf_a, buf_b, wt_v, b_v, acc_v, out_v, sem_a, sem_b, sem_w):
        wid = lax.axis_index("s") * 2 + lax.axis_index("c")
        row = base_row + wid
        bufs = (buf_a, buf_b)
        sems = (sem_a, sem_b)

        wcp = pltpu.async_copy(wt_hbm, wt_v, sem_w)
        pending = pltpu.async_copy(
            x_hbm.at[row, pl.ds(0, _CH), :], bufs[0], sems[0])
        for g in range(n_cgrp):
            acc_v[pl.ds(g * _LANES, _LANES)] = jnp.zeros(
                (_LANES,), jnp.float32)
        pltpu.sync_copy(b_hbm, b_v)
        wcp.wait()

        for c in range(n_chunks):
            nxt = None
            if c + 1 < n_chunks:
                nxt = pltpu.async_copy(
                    x_hbm.at[row, pl.ds((c + 1) * _CH, _CH), :],
                    bufs[(c + 1) % 2], sems[(c + 1) % 2])
            pending.wait()
            buf = bufs[c % 2]

            def acc_row(r, carry, buf=buf):
                for g in range(n_cgrp):
                    sl = pl.ds(g * _LANES, _LANES)
                    plsc.addupdate(acc_v.at[sl], buf[r, sl])
                return carry

            lax.fori_loop(0, _CH, acc_row, jnp.int32(0))
            pending = nxt

        def dot_g(g, carry):
            a = acc_v[pl.ds(g * _LANES, _LANES)]
            lvs = list(carry)
            for j in range(_LANES):
                s = a[j]
                ch = g * _LANES + j
                for ec in range(n_egrp):
                    lvs[ec] = lvs[ec] + s * wt_v[pl.ds(
                        ch * E + ec * _LANES, _LANES)]
            return tuple(lvs)

        zero = jnp.zeros((_LANES,), jnp.float32)
        lvs = lax.fori_loop(0, n_cgrp, dot_g, (zero,) * n_egrp)
        lvs = [lvs[ec] * (1.0 / HW) + b_v[pl.ds(ec * _LANES, _LANES)]
               for ec in range(n_egrp)]

        lane = lax.iota(jnp.int32, _LANES)

        def _xlane(v, op):
            for sh in (8, 4, 2, 1):
                p = jnp.bitwise_xor(lane, sh)
                v = op(v, v.at[p].get(mode="promise_in_bounds"))
            return v

        idxs = [lane + ec * _LANES for ec in range(n_egrp)]
        m = lvs[0]
        for ec in range(1, n_egrp):
            m = jnp.maximum(m, lvs[ec])
        m1 = _xlane(m, jnp.maximum)
        cand = jnp.where(lvs[0] == m1, idxs[0], E)
        for ec in range(1, n_egrp):
            cand = jnp.minimum(cand, jnp.where(lvs[ec] == m1, idxs[ec], E))
        i1 = _xlane(cand, jnp.minimum)
        neg = [jnp.where(idxs[ec] == i1, -jnp.inf, lvs[ec])
               for ec in range(n_egrp)]
        m2p = neg[0]
        for ec in range(1, n_egrp):
            m2p = jnp.maximum(m2p, neg[ec])
        m2 = _xlane(m2p, jnp.maximum)
        cand2 = jnp.where(neg[0] == m2, idxs[0], E)
        for ec in range(1, n_egrp):
            cand2 = jnp.minimum(cand2, jnp.where(neg[ec] == m2, idxs[ec], E))
        i2 = _xlane(cand2, jnp.minimum)
        e2 = jnp.exp(m2 - m1)
        denom = 1.0 + e2
        w1 = 1.0 / denom
        w2 = e2 / denom
        for ec in range(n_egrp):
            out_v[pl.ds(ec * _LANES, _LANES)] = jnp.where(
                idxs[ec] == i1, w1, jnp.where(idxs[ec] == i2, w2, 0.0))
        pltpu.sync_copy(out_v, out_hbm.at[wid])

    return sc_gate


@jax.jit
def kernel(x, W, b):
    B, C, H, Wd = x.shape
    E = W.shape[0]
    HW = H * Wd
    x3 = jnp.transpose(x, (0, 2, 3, 1)).reshape(B, HW, C)
    n_tc = B - _N_SC

    sc_out = _make_sc_gate(B, HW, C, E, n_tc)(x3, W.T.reshape(-1), b)

    b2 = b.reshape(1, E)
    BB = 8
    tc_out = pl.pallas_call(
        _tc_body,
        grid=(n_tc // BB,),
        in_specs=[
            pl.BlockSpec((BB, HW, C), lambda i: (i, 0, 0)),
            pl.BlockSpec((E, C), lambda i: (0, 0)),
            pl.BlockSpec((1, E), lambda i: (0, 0)),
        ],
        out_specs=pl.BlockSpec((BB, E), lambda i: (i, 0)),
        out_shape=jax.ShapeDtypeStruct((n_tc, E), jnp.float32),
    )(x3, W, b2)
    return jnp.concatenate([tc_out, sc_out], axis=0)


# speedup vs baseline: 1.5397x; 1.5397x over previous
import functools

import jax
import jax.numpy as jnp
from jax import lax
from jax.experimental import pallas as pl
from jax.experimental.pallas import tpu as pltpu
from jax.experimental.pallas import tpu_sc as plsc

_N_SC = 32
_CH = 32
_LANES = 16


def _tc_body(x_ref, w_ref, b_ref, out_ref):
    s = jnp.sum(x_ref[...], axis=1)
    pooled = s * (1.0 / x_ref.shape[1])
    logits = lax.dot_general(
        pooled, w_ref[...], (((1,), (1,)), ((), ())),
        preferred_element_type=jnp.float32,
    ) + b_ref[...]
    bb, e = logits.shape
    idx = lax.broadcasted_iota(jnp.int32, (bb, e), 1)
    m1 = jnp.max(logits, axis=1, keepdims=True)
    i1 = jnp.min(jnp.where(logits == m1, idx, e), axis=1, keepdims=True)
    neg = jnp.where(idx == i1, -jnp.inf, logits)
    m2 = jnp.max(neg, axis=1, keepdims=True)
    i2 = jnp.min(jnp.where(neg == m2, idx, e), axis=1, keepdims=True)
    e2 = jnp.exp(m2 - m1)
    denom = 1.0 + e2
    w1 = 1.0 / denom
    w2 = e2 / denom
    out_ref[...] = jnp.where(idx == i1, w1, jnp.where(idx == i2, w2, 0.0))


def _make_sc_gate(B, HW, C, E, base_row):
    n_chunks = HW // _CH
    n_cgrp = C // _LANES
    n_egrp = E // _LANES
    mesh = plsc.VectorSubcoreMesh(core_axis_name="c", subcore_axis_name="s")

    @functools.partial(
        pl.kernel,
        mesh=mesh,
        out_type=jax.ShapeDtypeStruct((_N_SC, E), jnp.float32),
        scratch_types=[
            pltpu.VMEM((_CH, C), jnp.float32),
            pltpu.VMEM((_CH, C), jnp.float32),
            pltpu.VMEM((C * E,), jnp.float32),
            pltpu.VMEM((E,), jnp.float32),
            pltpu.VMEM((C,), jnp.float32),
            pltpu.VMEM((E,), jnp.float32),
            pltpu.SemaphoreType.DMA,
            pltpu.SemaphoreType.DMA,
            pltpu.SemaphoreType.DMA,
        ],
    )
    def sc_gate(x_hbm, wt_hbm, b_hbm, out_hbm,
                buf_a, buf_b, wt_v, b_v, acc_v, out_v, sem_a, sem_b, sem_w):
        wid = lax.axis_index("s") * 2 + lax.axis_index("c")
        row = base_row + wid
        bufs = (buf_a, buf_b)
        sems = (sem_a, sem_b)

        wcp = pltpu.async_copy(wt_hbm, wt_v, sem_w)
        pending = pltpu.async_copy(
            x_hbm.at[row, pl.ds(0, _CH), :], bufs[0], sems[0])
        pltpu.sync_copy(b_hbm, b_v)
        wcp.wait()

        half = n_cgrp // 2
        zero = jnp.zeros((_LANES,), jnp.float32)
        accs = [(zero,) * half, (zero,) * half]
        for c in range(n_chunks):
            nxt = None
            if c + 1 < n_chunks:
                nxt = pltpu.async_copy(
                    x_hbm.at[row, pl.ds((c + 1) * _CH, _CH), :],
                    bufs[(c + 1) % 2], sems[(c + 1) % 2])
            pending.wait()
            buf = bufs[c % 2]
            for p in range(2):
                def acc_rows(r2, acc, buf=buf, p=p):
                    acc = list(acc)
                    for rr in range(2):
                        for g in range(half):
                            sl = pl.ds((p * half + g) * _LANES, _LANES)
                            acc[g] = acc[g] + buf[r2 * 2 + rr, sl]
                    return tuple(acc)

                accs[p] = lax.fori_loop(0, _CH // 2, acc_rows, accs[p])
            pending = nxt
        for p in range(2):
            for g in range(half):
                acc_v[pl.ds((p * half + g) * _LANES, _LANES)] = accs[p][g]

        def dot_g(g, carry):
            a = acc_v[pl.ds(g * _LANES, _LANES)]
            lvs = list(carry)
            for j in range(_LANES):
                s = a[j]
                ch = g * _LANES + j
                for ec in range(n_egrp):
                    lvs[ec] = lvs[ec] + s * wt_v[pl.ds(
                        ch * E + ec * _LANES, _LANES)]
            return tuple(lvs)

        zero = jnp.zeros((_LANES,), jnp.float32)
        lvs = lax.fori_loop(0, n_cgrp, dot_g, (zero,) * n_egrp)
        lvs = [lvs[ec] * (1.0 / HW) + b_v[pl.ds(ec * _LANES, _LANES)]
               for ec in range(n_egrp)]

        lane = lax.iota(jnp.int32, _LANES)

        def _xlane(v, op):
            for sh in (8, 4, 2, 1):
                p = jnp.bitwise_xor(lane, sh)
                v = op(v, v.at[p].get(mode="promise_in_bounds"))
            return v

        idxs = [lane + ec * _LANES for ec in range(n_egrp)]
        m = lvs[0]
        for ec in range(1, n_egrp):
            m = jnp.maximum(m, lvs[ec])
        m1 = _xlane(m, jnp.maximum)
        cand = jnp.where(lvs[0] == m1, idxs[0], E)
        for ec in range(1, n_egrp):
            cand = jnp.minimum(cand, jnp.where(lvs[ec] == m1, idxs[ec], E))
        i1 = _xlane(cand, jnp.minimum)
        neg = [jnp.where(idxs[ec] == i1, -jnp.inf, lvs[ec])
               for ec in range(n_egrp)]
        m2p = neg[0]
        for ec in range(1, n_egrp):
            m2p = jnp.maximum(m2p, neg[ec])
        m2 = _xlane(m2p, jnp.maximum)
        cand2 = jnp.where(neg[0] == m2, idxs[0], E)
        for ec in range(1, n_egrp):
            cand2 = jnp.minimum(cand2, jnp.where(neg[ec] == m2, idxs[ec], E))
        i2 = _xlane(cand2, jnp.minimum)
        e2 = jnp.exp(m2 - m1)
        denom = 1.0 + e2
        w1 = 1.0 / denom
        w2 = e2 / denom
        for ec in range(n_egrp):
            out_v[pl.ds(ec * _LANES, _LANES)] = jnp.where(
                idxs[ec] == i1, w1, jnp.where(idxs[ec] == i2, w2, 0.0))
        pltpu.sync_copy(out_v, out_hbm.at[wid])

    return sc_gate


@jax.jit
def kernel(x, W, b):
    B, C, H, Wd = x.shape
    E = W.shape[0]
    HW = H * Wd
    x3 = jnp.transpose(x, (0, 2, 3, 1)).reshape(B, HW, C)
    n_tc = B - _N_SC

    sc_out = _make_sc_gate(B, HW, C, E, n_tc)(x3, W.T.reshape(-1), b)

    b2 = b.reshape(1, E)
    BB = 8
    tc_out = pl.pallas_call(
        _tc_body,
        grid=(n_tc // BB,),
        in_specs=[
            pl.BlockSpec((BB, HW, C), lambda i: (i, 0, 0)),
            pl.BlockSpec((E, C), lambda i: (0, 0)),
            pl.BlockSpec((1, E), lambda i: (0, 0)),
        ],
        out_specs=pl.BlockSpec((BB, E), lambda i: (i, 0)),
        out_shape=jax.ShapeDtypeStruct((n_tc, E), jnp.float32),
    )(x3, W, b2)
    return jnp.concatenate([tc_out, sc_out], axis=0)


# speedup vs baseline: 2.0206x; 1.3123x over previous
import functools

import jax
import jax.numpy as jnp
from jax import lax
from jax.experimental import pallas as pl
from jax.experimental.pallas import tpu as pltpu
from jax.experimental.pallas import tpu_sc as plsc

_N_SC = 0
_CH = 32
_LANES = 16


def _tc_body(x_ref, w_ref, b_ref, out_ref):
    s = jnp.sum(x_ref[...], axis=1)
    pooled = s * (1.0 / x_ref.shape[1])
    logits = lax.dot_general(
        pooled, w_ref[...], (((1,), (1,)), ((), ())),
        preferred_element_type=jnp.float32,
    ) + b_ref[...]
    bb, e = logits.shape
    idx = lax.broadcasted_iota(jnp.int32, (bb, e), 1)
    m1 = jnp.max(logits, axis=1, keepdims=True)
    i1 = jnp.min(jnp.where(logits == m1, idx, e), axis=1, keepdims=True)
    neg = jnp.where(idx == i1, -jnp.inf, logits)
    m2 = jnp.max(neg, axis=1, keepdims=True)
    i2 = jnp.min(jnp.where(neg == m2, idx, e), axis=1, keepdims=True)
    e2 = jnp.exp(m2 - m1)
    denom = 1.0 + e2
    w1 = 1.0 / denom
    w2 = e2 / denom
    out_ref[...] = jnp.where(idx == i1, w1, jnp.where(idx == i2, w2, 0.0))


def _make_sc_gate(B, HW, C, E, base_row):
    n_chunks = HW // _CH
    n_cgrp = C // _LANES
    n_egrp = E // _LANES
    mesh = plsc.VectorSubcoreMesh(core_axis_name="c", subcore_axis_name="s")

    @functools.partial(
        pl.kernel,
        mesh=mesh,
        out_type=jax.ShapeDtypeStruct((_N_SC, E), jnp.float32),
        scratch_types=[
            pltpu.VMEM((_CH, C), jnp.float32),
            pltpu.VMEM((_CH, C), jnp.float32),
            pltpu.VMEM((C * E,), jnp.float32),
            pltpu.VMEM((E,), jnp.float32),
            pltpu.VMEM((C,), jnp.float32),
            pltpu.VMEM((E,), jnp.float32),
            pltpu.SemaphoreType.DMA,
            pltpu.SemaphoreType.DMA,
            pltpu.SemaphoreType.DMA,
        ],
    )
    def sc_gate(x_hbm, wt_hbm, b_hbm, out_hbm,
                buf_a, buf_b, wt_v, b_v, acc_v, out_v, sem_a, sem_b, sem_w):
        wid = lax.axis_index("s") * 2 + lax.axis_index("c")
        row = base_row + wid
        bufs = (buf_a, buf_b)
        sems = (sem_a, sem_b)

        wcp = pltpu.async_copy(wt_hbm, wt_v, sem_w)
        pending = pltpu.async_copy(
            x_hbm.at[row, pl.ds(0, _CH), :], bufs[0], sems[0])
        pltpu.sync_copy(b_hbm, b_v)
        wcp.wait()

        half = n_cgrp // 2
        zero = jnp.zeros((_LANES,), jnp.float32)
        accs = [(zero,) * half, (zero,) * half]
        for c in range(n_chunks):
            nxt = None
            if c + 1 < n_chunks:
                nxt = pltpu.async_copy(
                    x_hbm.at[row, pl.ds((c + 1) * _CH, _CH), :],
                    bufs[(c + 1) % 2], sems[(c + 1) % 2])
            pending.wait()
            buf = bufs[c % 2]
            for p in range(2):
                def acc_rows(r2, acc, buf=buf, p=p):
                    acc = list(acc)
                    for rr in range(2):
                        for g in range(half):
                            sl = pl.ds((p * half + g) * _LANES, _LANES)
                            acc[g] = acc[g] + buf[r2 * 2 + rr, sl]
                    return tuple(acc)

                accs[p] = lax.fori_loop(0, _CH // 2, acc_rows, accs[p])
            pending = nxt
        for p in range(2):
            for g in range(half):
                acc_v[pl.ds((p * half + g) * _LANES, _LANES)] = accs[p][g]

        def dot_g(g, carry):
            a = acc_v[pl.ds(g * _LANES, _LANES)]
            lvs = list(carry)
            for j in range(_LANES):
                s = a[j]
                ch = g * _LANES + j
                for ec in range(n_egrp):
                    lvs[ec] = lvs[ec] + s * wt_v[pl.ds(
                        ch * E + ec * _LANES, _LANES)]
            return tuple(lvs)

        zero = jnp.zeros((_LANES,), jnp.float32)
        lvs = lax.fori_loop(0, n_cgrp, dot_g, (zero,) * n_egrp)
        lvs = [lvs[ec] * (1.0 / HW) + b_v[pl.ds(ec * _LANES, _LANES)]
               for ec in range(n_egrp)]

        lane = lax.iota(jnp.int32, _LANES)

        def _xlane(v, op):
            for sh in (8, 4, 2, 1):
                p = jnp.bitwise_xor(lane, sh)
                v = op(v, v.at[p].get(mode="promise_in_bounds"))
            return v

        idxs = [lane + ec * _LANES for ec in range(n_egrp)]
        m = lvs[0]
        for ec in range(1, n_egrp):
            m = jnp.maximum(m, lvs[ec])
        m1 = _xlane(m, jnp.maximum)
        cand = jnp.where(lvs[0] == m1, idxs[0], E)
        for ec in range(1, n_egrp):
            cand = jnp.minimum(cand, jnp.where(lvs[ec] == m1, idxs[ec], E))
        i1 = _xlane(cand, jnp.minimum)
        neg = [jnp.where(idxs[ec] == i1, -jnp.inf, lvs[ec])
               for ec in range(n_egrp)]
        m2p = neg[0]
        for ec in range(1, n_egrp):
            m2p = jnp.maximum(m2p, neg[ec])
        m2 = _xlane(m2p, jnp.maximum)
        cand2 = jnp.where(neg[0] == m2, idxs[0], E)
        for ec in range(1, n_egrp):
            cand2 = jnp.minimum(cand2, jnp.where(neg[ec] == m2, idxs[ec], E))
        i2 = _xlane(cand2, jnp.minimum)
        e2 = jnp.exp(m2 - m1)
        denom = 1.0 + e2
        w1 = 1.0 / denom
        w2 = e2 / denom
        for ec in range(n_egrp):
            out_v[pl.ds(ec * _LANES, _LANES)] = jnp.where(
                idxs[ec] == i1, w1, jnp.where(idxs[ec] == i2, w2, 0.0))
        pltpu.sync_copy(out_v, out_hbm.at[wid])

    return sc_gate


@jax.jit
def kernel(x, W, b):
    B, C, H, Wd = x.shape
    E = W.shape[0]
    HW = H * Wd
    x3 = jnp.transpose(x, (0, 2, 3, 1)).reshape(B, HW, C)
    n_tc = B - _N_SC

    if _N_SC:
        sc_out = _make_sc_gate(B, HW, C, E, n_tc)(x3, W.T.reshape(-1), b)

    b2 = b.reshape(1, E)
    BB = 16
    tc_out = pl.pallas_call(
        _tc_body,
        grid=(n_tc // BB,),
        in_specs=[
            pl.BlockSpec((BB, HW, C), lambda i: (i, 0, 0)),
            pl.BlockSpec((E, C), lambda i: (0, 0)),
            pl.BlockSpec((1, E), lambda i: (0, 0)),
        ],
        out_specs=pl.BlockSpec((BB, E), lambda i: (i, 0)),
        out_shape=jax.ShapeDtypeStruct((n_tc, E), jnp.float32),
    )(x3, W, b2)
    if not _N_SC:
        return tc_out
    return jnp.concatenate([tc_out, sc_out], axis=0)


# speedup vs baseline: 2.0405x; 1.0099x over previous
import functools

import jax
import jax.numpy as jnp
from jax import lax
from jax.experimental import pallas as pl
from jax.experimental.pallas import tpu as pltpu
from jax.experimental.pallas import tpu_sc as plsc

_N_SC = 0
_CH = 32
_LANES = 16


def _tc_body(x_ref, w_ref, b_ref, out_ref):
    s = jnp.sum(x_ref[...], axis=1)
    pooled = s * (1.0 / x_ref.shape[1])
    logits = lax.dot_general(
        pooled, w_ref[...], (((1,), (1,)), ((), ())),
        preferred_element_type=jnp.float32,
    ) + b_ref[...]
    bb, e = logits.shape
    idx = lax.broadcasted_iota(jnp.int32, (bb, e), 1)
    m1 = jnp.max(logits, axis=1, keepdims=True)
    i1 = jnp.min(jnp.where(logits == m1, idx, e), axis=1, keepdims=True)
    neg = jnp.where(idx == i1, -jnp.inf, logits)
    m2 = jnp.max(neg, axis=1, keepdims=True)
    i2 = jnp.min(jnp.where(neg == m2, idx, e), axis=1, keepdims=True)
    e2 = jnp.exp(m2 - m1)
    denom = 1.0 + e2
    w1 = 1.0 / denom
    w2 = e2 / denom
    out_ref[...] = jnp.where(idx == i1, w1, jnp.where(idx == i2, w2, 0.0))


def _make_sc_gate(B, HW, C, E, base_row):
    n_chunks = HW // _CH
    n_cgrp = C // _LANES
    n_egrp = E // _LANES
    mesh = plsc.VectorSubcoreMesh(core_axis_name="c", subcore_axis_name="s")

    @functools.partial(
        pl.kernel,
        mesh=mesh,
        out_type=jax.ShapeDtypeStruct((_N_SC, E), jnp.float32),
        scratch_types=[
            pltpu.VMEM((_CH, C), jnp.float32),
            pltpu.VMEM((_CH, C), jnp.float32),
            pltpu.VMEM((C * E,), jnp.float32),
            pltpu.VMEM((E,), jnp.float32),
            pltpu.VMEM((C,), jnp.float32),
            pltpu.VMEM((E,), jnp.float32),
            pltpu.SemaphoreType.DMA,
            pltpu.SemaphoreType.DMA,
            pltpu.SemaphoreType.DMA,
        ],
    )
    def sc_gate(x_hbm, wt_hbm, b_hbm, out_hbm,
                buf_a, buf_b, wt_v, b_v, acc_v, out_v, sem_a, sem_b, sem_w):
        wid = lax.axis_index("s") * 2 + lax.axis_index("c")
        row = base_row + wid
        bufs = (buf_a, buf_b)
        sems = (sem_a, sem_b)

        wcp = pltpu.async_copy(wt_hbm, wt_v, sem_w)
        pending = pltpu.async_copy(
            x_hbm.at[row, pl.ds(0, _CH), :], bufs[0], sems[0])
        pltpu.sync_copy(b_hbm, b_v)
        wcp.wait()

        half = n_cgrp // 2
        zero = jnp.zeros((_LANES,), jnp.float32)
        accs = [(zero,) * half, (zero,) * half]
        for c in range(n_chunks):
            nxt = None
            if c + 1 < n_chunks:
                nxt = pltpu.async_copy(
                    x_hbm.at[row, pl.ds((c + 1) * _CH, _CH), :],
                    bufs[(c + 1) % 2], sems[(c + 1) % 2])
            pending.wait()
            buf = bufs[c % 2]
            for p in range(2):
                def acc_rows(r2, acc, buf=buf, p=p):
                    acc = list(acc)
                    for rr in range(2):
                        for g in range(half):
                            sl = pl.ds((p * half + g) * _LANES, _LANES)
                            acc[g] = acc[g] + buf[r2 * 2 + rr, sl]
                    return tuple(acc)

                accs[p] = lax.fori_loop(0, _CH // 2, acc_rows, accs[p])
            pending = nxt
        for p in range(2):
            for g in range(half):
                acc_v[pl.ds((p * half + g) * _LANES, _LANES)] = accs[p][g]

        def dot_g(g, carry):
            a = acc_v[pl.ds(g * _LANES, _LANES)]
            lvs = list(carry)
            for j in range(_LANES):
                s = a[j]
                ch = g * _LANES + j
                for ec in range(n_egrp):
                    lvs[ec] = lvs[ec] + s * wt_v[pl.ds(
                        ch * E + ec * _LANES, _LANES)]
            return tuple(lvs)

        zero = jnp.zeros((_LANES,), jnp.float32)
        lvs = lax.fori_loop(0, n_cgrp, dot_g, (zero,) * n_egrp)
        lvs = [lvs[ec] * (1.0 / HW) + b_v[pl.ds(ec * _LANES, _LANES)]
               for ec in range(n_egrp)]

        lane = lax.iota(jnp.int32, _LANES)

        def _xlane(v, op):
            for sh in (8, 4, 2, 1):
                p = jnp.bitwise_xor(lane, sh)
                v = op(v, v.at[p].get(mode="promise_in_bounds"))
            return v

        idxs = [lane + ec * _LANES for ec in range(n_egrp)]
        m = lvs[0]
        for ec in range(1, n_egrp):
            m = jnp.maximum(m, lvs[ec])
        m1 = _xlane(m, jnp.maximum)
        cand = jnp.where(lvs[0] == m1, idxs[0], E)
        for ec in range(1, n_egrp):
            cand = jnp.minimum(cand, jnp.where(lvs[ec] == m1, idxs[ec], E))
        i1 = _xlane(cand, jnp.minimum)
        neg = [jnp.where(idxs[ec] == i1, -jnp.inf, lvs[ec])
               for ec in range(n_egrp)]
        m2p = neg[0]
        for ec in range(1, n_egrp):
            m2p = jnp.maximum(m2p, neg[ec])
        m2 = _xlane(m2p, jnp.maximum)
        cand2 = jnp.where(neg[0] == m2, idxs[0], E)
        for ec in range(1, n_egrp):
            cand2 = jnp.minimum(cand2, jnp.where(neg[ec] == m2, idxs[ec], E))
        i2 = _xlane(cand2, jnp.minimum)
        e2 = jnp.exp(m2 - m1)
        denom = 1.0 + e2
        w1 = 1.0 / denom
        w2 = e2 / denom
        for ec in range(n_egrp):
            out_v[pl.ds(ec * _LANES, _LANES)] = jnp.where(
                idxs[ec] == i1, w1, jnp.where(idxs[ec] == i2, w2, 0.0))
        pltpu.sync_copy(out_v, out_hbm.at[wid])

    return sc_gate


@jax.jit
def kernel(x, W, b):
    B, C, H, Wd = x.shape
    E = W.shape[0]
    HW = H * Wd
    x3 = jnp.transpose(x, (0, 2, 3, 1)).reshape(B, HW, C)
    n_tc = B - _N_SC

    if _N_SC:
        sc_out = _make_sc_gate(B, HW, C, E, n_tc)(x3, W.T.reshape(-1), b)

    b2 = b.reshape(1, E)
    BB = 8
    tc_out = pl.pallas_call(
        _tc_body,
        grid=(n_tc // BB,),
        in_specs=[
            pl.BlockSpec((BB, HW, C), lambda i: (i, 0, 0)),
            pl.BlockSpec((E, C), lambda i: (0, 0)),
            pl.BlockSpec((1, E), lambda i: (0, 0)),
        ],
        out_specs=pl.BlockSpec((BB, E), lambda i: (i, 0)),
        out_shape=jax.ShapeDtypeStruct((n_tc, E), jnp.float32),
    )(x3, W, b2)
    if not _N_SC:
        return tc_out
    return jnp.concatenate([tc_out, sc_out], axis=0)


# speedup vs baseline: 2.0517x; 1.0055x over previous
import jax
import jax.numpy as jnp
from jax import lax
from jax.experimental import pallas as pl


def _body(x_ref, w_ref, b_ref, out_ref):
    s = jnp.sum(x_ref[...], axis=1)
    pooled = s * (1.0 / x_ref.shape[1])
    logits = lax.dot_general(
        pooled, w_ref[...], (((1,), (1,)), ((), ())),
        preferred_element_type=jnp.float32,
    ) + b_ref[...]
    bb, e = logits.shape
    idx = lax.broadcasted_iota(jnp.int32, (bb, e), 1)
    m1 = jnp.max(logits, axis=1, keepdims=True)
    i1 = jnp.min(jnp.where(logits == m1, idx, e), axis=1, keepdims=True)
    neg = jnp.where(idx == i1, -jnp.inf, logits)
    m2 = jnp.max(neg, axis=1, keepdims=True)
    i2 = jnp.min(jnp.where(neg == m2, idx, e), axis=1, keepdims=True)
    e2 = jnp.exp(m2 - m1)
    denom = 1.0 + e2
    w1 = 1.0 / denom
    w2 = e2 / denom
    out_ref[...] = jnp.where(idx == i1, w1, jnp.where(idx == i2, w2, 0.0))


@jax.jit
def kernel(x, W, b):
    B, C, H, Wd = x.shape
    E = W.shape[0]
    HW = H * Wd
    x3 = jnp.transpose(x, (0, 2, 3, 1)).reshape(B, HW, C)
    b2 = b.reshape(1, E)
    BB = 8
    return pl.pallas_call(
        _body,
        grid=(B // BB,),
        in_specs=[
            pl.BlockSpec((BB, HW, C), lambda i: (i, 0, 0)),
            pl.BlockSpec((E, C), lambda i: (0, 0)),
            pl.BlockSpec((1, E), lambda i: (0, 0)),
        ],
        out_specs=pl.BlockSpec((BB, E), lambda i: (i, 0)),
        out_shape=jax.ShapeDtypeStruct((B, E), jnp.float32),
    )(x3, W, b2)
